# baseline (speedup 1.0000x reference)
import jax
import jax.numpy as jnp
from jax import lax
from jax.experimental import pallas as pl
from jax.experimental.pallas import tpu as pltpu
from jax.experimental.pallas import tpu_sc as plsc

N_INST = 20000
D_FEAT = 1024
D_HID = 256
K_SEL = N_INST // 2

BN = 2000
NT = 16
CHUNK = 1280
NV = CHUNK // 16
TAIL = N_INST - (NT - 1) * CHUNK

_I32MAX_PY = 0x7FFFFFFF


def _mlp_kernel(x_ref, w1_ref, b1_ref, w2_ref, b2_ref, out_ref):
    h = jnp.dot(x_ref[...], w1_ref[...], preferred_element_type=jnp.float32)
    h = jnp.maximum(h + b1_ref[...], 0.0)
    out_ref[...] = (
        jnp.dot(h, w2_ref[...], preferred_element_type=jnp.float32) + b2_ref[...]
    )


def _sc_select(scores_hbm, topv_hbm, out_hbm,
               chunk_f, chunk_v, topv_v, merged_v, gh_v, crow_v, cnts_v,
               res_v, out_v, sh_hist, sh_cnts, sh_res):
    core = lax.axis_index("c")
    tid = lax.axis_index("s")

    lane = lax.iota(jnp.int32, 16)
    ones16 = jnp.ones((16,), jnp.int32)
    z16 = jnp.zeros((16,), jnp.int32)
    kK = jnp.int32(K_SEL)

    @pl.when(core == 0)
    def _body():
        base = tid * CHUNK

        @pl.when(tid < NT - 1)
        def _full():
            pltpu.sync_copy(scores_hbm.at[pl.ds(base, CHUNK)], chunk_f)

        @pl.when(tid == NT - 1)
        def _tail():
            pltpu.sync_copy(scores_hbm.at[pl.ds((NT - 1) * CHUNK, TAIL)],
                            chunk_f.at[pl.ds(0, TAIL)])

        pltpu.sync_copy(topv_hbm, topv_v)

        def to_sortable(j, _):
            b = plsc.bitcast(chunk_f[pl.ds(j * 16, 16)], jnp.int32)
            chunk_v[pl.ds(j * 16, 16)] = b ^ ((b >> 31) & jnp.int32(_I32MAX_PY))
            return 0

        lax.fori_loop(0, NV, to_sortable, 0)

        def pad_mask(j):
            return (base + j * 16 + lane) < jnp.int32(N_INST)

        def locate(a_rank, half):
            def scan16(j, carry):
                csum, bstar, cbelow, found = carry
                acc = z16
                for t in range(NT):
                    acc = acc + gh_v[pl.ds(t * 512 + half * 256 + j * 16, 16)]
                inc = plsc.cumsum(acc) + csum
                excl = inc - acc
                m = inc >= a_rank
                first = jnp.min(jnp.where(m, lane, jnp.int32(16)))
                cb = jnp.min(jnp.where(m, excl, jnp.int32(_I32MAX_PY)))
                newly = (first < 16) & (found == 0)
                bstar = jnp.where(newly, j * 16 + first, bstar)
                cbelow = jnp.where(newly, cb, cbelow)
                found = jnp.where(first < 16, jnp.int32(1), found)
                csum = jnp.max(inc)
                return csum, bstar, cbelow, found

            _, bstar, cbelow, _ = lax.fori_loop(
                0, 16, scan16,
                (jnp.int32(0), jnp.int32(0), jnp.int32(0), jnp.int32(0)))
            return bstar, cbelow

        pA = jnp.int32(0)
        pB = jnp.int32(0)
        aA = kK + 1
        aB = kK

        for r, shift in enumerate((24, 16, 8, 0)):
            pmask = jnp.int32(0 if r == 0 else -(1 << (shift + 8)))

            def zero_row(j, _):
                gh_v[pl.ds(j * 16, 16)] = z16
                return 0

            lax.fori_loop(0, 512, zero_row, 0)

            def scan_body(j, _, pA=pA, pB=pB, pmask=pmask, shift=shift, r=r):
                s = chunk_v[pl.ds(j * 16, 16)]
                pm = pad_mask(j)
                bucket = (s >> shift) & jnp.int32(255)
                if r == 0:
                    bucket = bucket ^ jnp.int32(128)
                okA = ((s & pmask) == pA) & pm
                okB = ((s & pmask) == pB) & pm
                idx = lane * 512 + bucket
                plsc.addupdate_scatter(gh_v, [idx], ones16, mask=okA)
                plsc.addupdate_scatter(gh_v, [idx + 256], ones16, mask=okB)
                return 0

            lax.fori_loop(0, NV, scan_body, 0)

            def merge_body(j, _):
                acc = z16
                for l in range(16):
                    acc = acc + gh_v[pl.ds(l * 512 + j * 16, 16)]
                merged_v[pl.ds(j * 16, 16)] = acc
                return 0

            lax.fori_loop(0, 32, merge_body, 0)

            pltpu.sync_copy(merged_v, sh_hist.at[pl.ds(tid * 512, 512)])
            plsc.subcore_barrier()

            @pl.when(tid == 0)
            def _locate(pA=pA, pB=pB, aA=aA, aB=aB, shift=shift, r=r):
                pltpu.sync_copy(sh_hist, gh_v)
                bA, cbA = locate(aA, 0)
                bB, cbB = locate(aB, 1)
                if r == 0:
                    bA = bA ^ jnp.int32(128)
                    bB = bB ^ jnp.int32(128)
                npA = pA | (bA << shift)
                npB = pB | (bB << shift)
                crow_v[...] = jnp.where(lane == 0, npA,
                              jnp.where(lane == 1, npB,
                              jnp.where(lane == 2, aA - cbA,
                              jnp.where(lane == 3, aB - cbB, jnp.int32(0)))))
                pltpu.sync_copy(crow_v, sh_res)

            plsc.subcore_barrier()
            pltpu.sync_copy(sh_res, res_v)
            rv = res_v[...]
            pA = rv[0]
            pB = rv[1]
            aA = rv[2]
            aB = rv[3]

        T = pA
        T2 = pB

        def count_body(j, carry):
            g, e, l2, e2 = carry
            s = chunk_v[pl.ds(j * 16, 16)]
            pm = pad_mask(j)
            g = g + jnp.where((s > T) & pm, 1, 0)
            e = e + jnp.where((s == T) & pm, 1, 0)
            l2 = l2 + jnp.where((s < T2) & pm, 1, 0)
            e2 = e2 + jnp.where((s == T2) & pm, 1, 0)
            return g, e, l2, e2

        g, e, l2, e2 = lax.fori_loop(0, NV, count_body, (z16, z16, z16, z16))
        crow_v[...] = jnp.where(lane == 0, jnp.sum(g),
                      jnp.where(lane == 1, jnp.sum(e),
                      jnp.where(lane == 2, jnp.sum(l2),
                      jnp.where(lane == 3, jnp.sum(e2), jnp.int32(0)))))
        pltpu.sync_copy(crow_v, sh_cnts.at[pl.ds(tid * 16, 16)])
        plsc.subcore_barrier()
        pltpu.sync_copy(sh_cnts, cnts_v)

        tot = z16
        pref = z16
        for t in range(NT):
            rt = cnts_v[pl.ds(t * 16, 16)]
            tot = tot + rt
            pref = pref + jnp.where(jnp.int32(t) < tid, rt, 0)

        def lane_at(v, k):
            return jnp.sum(jnp.where(lane == k, v, 0))

        G = lane_at(tot, 0)
        L = lane_at(tot, 2)
        my_prefT = lane_at(pref, 1)
        my_prefT2 = lane_at(pref, 3)

        limT = kK - G
        limT2 = kK - L
        topv = topv_v[...]

        def label_body(j, carry):
            pT_run, pT2_run = carry
            s = chunk_v[pl.ds(j * 16, 16)]
            meT = (s == T)
            meT2 = (s == T2)
            ceT = plsc.cumsum(jnp.where(meT, 1, 0))
            ceT2 = plsc.cumsum(jnp.where(meT2, 1, 0))
            rT = pT_run + ceT - jnp.where(meT, 1, 0)
            rT2 = pT2_run + ceT2 - jnp.where(meT2, 1, 0)
            in_top = (s > T) | (meT & (rT < limT))
            in_bot = (s < T2) | (meT2 & (rT2 < limT2))
            out_v[pl.ds(j * 16, 16)] = jnp.where(
                in_top & (~in_bot), topv, jnp.float32(0.0))
            return pT_run + jnp.max(ceT), pT2_run + jnp.max(ceT2)

        lax.fori_loop(0, NV, label_body, (my_prefT + z16, my_prefT2 + z16))

        @pl.when(tid < NT - 1)
        def _wfull():
            pltpu.sync_copy(out_v, out_hbm.at[pl.ds(base, CHUNK)])

        @pl.when(tid == NT - 1)
        def _wtail():
            pltpu.sync_copy(out_v.at[pl.ds(0, TAIL)],
                            out_hbm.at[pl.ds((NT - 1) * CHUNK, TAIL)])


@jax.jit
def _run(instances, bag_label, W1, b1, W2, b2):
    x = instances[0]
    preds = pl.pallas_call(
        _mlp_kernel,
        grid=(N_INST // BN,),
        in_specs=[
            pl.BlockSpec((BN, D_FEAT), lambda i: (i, 0)),
            pl.BlockSpec((D_FEAT, D_HID), lambda i: (0, 0)),
            pl.BlockSpec((1, D_HID), lambda i: (0, 0)),
            pl.BlockSpec((D_HID, 1), lambda i: (0, 0)),
            pl.BlockSpec((1, 1), lambda i: (0, 0)),
        ],
        out_specs=pl.BlockSpec((BN, 1), lambda i: (i, 0)),
        out_shape=jax.ShapeDtypeStruct((N_INST, 1), jnp.float32),
    )(x, W1, b1.reshape(1, D_HID), W2, b2.reshape(1, 1))

    top_val = jnp.where(bag_label[0] != 0.0, jnp.float32(1.0), jnp.float32(0.0))
    topv = jnp.broadcast_to(top_val, (16,))

    mesh = plsc.VectorSubcoreMesh(core_axis_name="c", subcore_axis_name="s")
    sel = pl.kernel(
        _sc_select,
        mesh=mesh,
        compiler_params=pltpu.CompilerParams(needs_layout_passes=False),
        out_type=jax.ShapeDtypeStruct((N_INST,), jnp.float32),
        scratch_types=[
            pltpu.VMEM((CHUNK,), jnp.float32),
            pltpu.VMEM((CHUNK,), jnp.int32),
            pltpu.VMEM((16,), jnp.float32),
            pltpu.VMEM((512,), jnp.int32),
            pltpu.VMEM((NT * 512,), jnp.int32),
            pltpu.VMEM((16,), jnp.int32),
            pltpu.VMEM((NT * 16,), jnp.int32),
            pltpu.VMEM((16,), jnp.int32),
            pltpu.VMEM((CHUNK,), jnp.float32),
            pltpu.VMEM_SHARED((NT * 512,), jnp.int32),
            pltpu.VMEM_SHARED((NT * 16,), jnp.int32),
            pltpu.VMEM_SHARED((16,), jnp.int32),
        ],
    )
    labels = sel(preds[:, 0], topv)

    return preds[None, ...], labels[:, None][None, ...]


def kernel(instances, bag_label, W1, b1, W2, b2):
    return _run(instances, bag_label, W1, b1, W2, b2)


# speedup vs baseline: 1.0793x; 1.0793x over previous
import jax
import jax.numpy as jnp
from jax import lax
from jax.experimental import pallas as pl
from jax.experimental.pallas import tpu as pltpu
from jax.experimental.pallas import tpu_sc as plsc

N_INST = 20000
D_FEAT = 1024
D_HID = 256
K_SEL = N_INST // 2

BN = 2000
NT = 16
CHUNK = 1280
NV = CHUNK // 16
TAIL = N_INST - (NT - 1) * CHUNK

_I32MAX_PY = 0x7FFFFFFF


def _mlp_kernel(x_ref, w1_ref, b1_ref, w2_ref, b2_ref, out_ref):
    h = jnp.dot(x_ref[...], w1_ref[...], preferred_element_type=jnp.float32)
    h = jnp.maximum(h + b1_ref[...], 0.0)
    out_ref[...] = (
        jnp.dot(h, w2_ref[...], preferred_element_type=jnp.float32) + b2_ref[...]
    )


def _sc_select(scores_hbm, topv_hbm, out_hbm,
               chunk_f, chunk_v, topv_v, merged_v, gh_v, crow_v, cnts_v,
               res_v, out_v, sh_hist, sh_cnts, sh_res):
    core = lax.axis_index("c")
    tid = lax.axis_index("s")

    lane = lax.iota(jnp.int32, 16)
    ones16 = jnp.ones((16,), jnp.int32)
    z16 = jnp.zeros((16,), jnp.int32)
    kK = jnp.int32(K_SEL)

    @pl.when(core == 0)
    def _body():
        base = tid * CHUNK

        @pl.when(tid < NT - 1)
        def _full():
            pltpu.sync_copy(scores_hbm.at[pl.ds(base, CHUNK)], chunk_f)

        @pl.when(tid == NT - 1)
        def _tail():
            pltpu.sync_copy(scores_hbm.at[pl.ds((NT - 1) * CHUNK, TAIL)],
                            chunk_f.at[pl.ds(0, TAIL)])

        pltpu.sync_copy(topv_hbm, topv_v)

        def to_sortable(j, _):
            b = plsc.bitcast(chunk_f[pl.ds(j * 16, 16)], jnp.int32)
            chunk_v[pl.ds(j * 16, 16)] = b ^ ((b >> 31) & jnp.int32(_I32MAX_PY))
            return 0

        lax.fori_loop(0, NV, to_sortable, 0, unroll=8)

        def pad_mask(j):
            return (base + j * 16 + lane) < jnp.int32(N_INST)

        def locate(a_rank, half):
            def scan16(j, carry):
                csum, bstar, cbelow, found = carry
                acc = z16
                for t in range(NT):
                    acc = acc + gh_v[pl.ds(t * 512 + half * 256 + j * 16, 16)]
                inc = plsc.cumsum(acc) + csum
                excl = inc - acc
                m = inc >= a_rank
                first = jnp.min(jnp.where(m, lane, jnp.int32(16)))
                cb = jnp.min(jnp.where(m, excl, jnp.int32(_I32MAX_PY)))
                newly = (first < 16) & (found == 0)
                bstar = jnp.where(newly, j * 16 + first, bstar)
                cbelow = jnp.where(newly, cb, cbelow)
                found = jnp.where(first < 16, jnp.int32(1), found)
                csum = jnp.max(inc)
                return csum, bstar, cbelow, found

            _, bstar, cbelow, _ = lax.fori_loop(
                0, 16, scan16,
                (jnp.int32(0), jnp.int32(0), jnp.int32(0), jnp.int32(0)))
            return bstar, cbelow

        pA = jnp.int32(0)
        pB = jnp.int32(0)
        aA = kK + 1
        aB = kK

        for r, shift in enumerate((24, 16, 8, 0)):
            pmask = jnp.int32(0 if r == 0 else -(1 << (shift + 8)))

            def zero_row(j, _):
                gh_v[pl.ds(j * 16, 16)] = z16
                return 0

            lax.fori_loop(0, 512, zero_row, 0, unroll=8)

            def scan_body(j, _, pA=pA, pB=pB, pmask=pmask, shift=shift, r=r):
                s = chunk_v[pl.ds(j * 16, 16)]
                pm = pad_mask(j)
                bucket = (s >> shift) & jnp.int32(255)
                if r == 0:
                    bucket = bucket ^ jnp.int32(128)
                okA = ((s & pmask) == pA) & pm
                okB = ((s & pmask) == pB) & pm
                idx = lane * 512 + bucket
                plsc.addupdate_scatter(gh_v, [idx], ones16, mask=okA)
                plsc.addupdate_scatter(gh_v, [idx + 256], ones16, mask=okB)
                return 0

            lax.fori_loop(0, NV, scan_body, 0, unroll=4)

            def merge_body(j, _):
                acc = z16
                for l in range(16):
                    acc = acc + gh_v[pl.ds(l * 512 + j * 16, 16)]
                merged_v[pl.ds(j * 16, 16)] = acc
                return 0

            lax.fori_loop(0, 32, merge_body, 0, unroll=4)

            pltpu.sync_copy(merged_v, sh_hist.at[pl.ds(tid * 512, 512)])
            plsc.subcore_barrier()

            @pl.when(tid == 0)
            def _locate(pA=pA, pB=pB, aA=aA, aB=aB, shift=shift, r=r):
                pltpu.sync_copy(sh_hist, gh_v)
                bA, cbA = locate(aA, 0)
                bB, cbB = locate(aB, 1)
                if r == 0:
                    bA = bA ^ jnp.int32(128)
                    bB = bB ^ jnp.int32(128)
                npA = pA | (bA << shift)
                npB = pB | (bB << shift)
                crow_v[...] = jnp.where(lane == 0, npA,
                              jnp.where(lane == 1, npB,
                              jnp.where(lane == 2, aA - cbA,
                              jnp.where(lane == 3, aB - cbB, jnp.int32(0)))))
                pltpu.sync_copy(crow_v, sh_res)

            plsc.subcore_barrier()
            pltpu.sync_copy(sh_res, res_v)
            rv = res_v[...]
            pA = rv[0]
            pB = rv[1]
            aA = rv[2]
            aB = rv[3]

        T = pA
        T2 = pB

        def count_body(j, carry):
            g, e, l2, e2 = carry
            s = chunk_v[pl.ds(j * 16, 16)]
            pm = pad_mask(j)
            g = g + jnp.where((s > T) & pm, 1, 0)
            e = e + jnp.where((s == T) & pm, 1, 0)
            l2 = l2 + jnp.where((s < T2) & pm, 1, 0)
            e2 = e2 + jnp.where((s == T2) & pm, 1, 0)
            return g, e, l2, e2

        g, e, l2, e2 = lax.fori_loop(0, NV, count_body, (z16, z16, z16, z16), unroll=4)
        crow_v[...] = jnp.where(lane == 0, jnp.sum(g),
                      jnp.where(lane == 1, jnp.sum(e),
                      jnp.where(lane == 2, jnp.sum(l2),
                      jnp.where(lane == 3, jnp.sum(e2), jnp.int32(0)))))
        pltpu.sync_copy(crow_v, sh_cnts.at[pl.ds(tid * 16, 16)])
        plsc.subcore_barrier()
        pltpu.sync_copy(sh_cnts, cnts_v)

        tot = z16
        pref = z16
        for t in range(NT):
            rt = cnts_v[pl.ds(t * 16, 16)]
            tot = tot + rt
            pref = pref + jnp.where(jnp.int32(t) < tid, rt, 0)

        def lane_at(v, k):
            return jnp.sum(jnp.where(lane == k, v, 0))

        G = lane_at(tot, 0)
        L = lane_at(tot, 2)
        my_prefT = lane_at(pref, 1)
        my_prefT2 = lane_at(pref, 3)

        limT = kK - G
        limT2 = kK - L
        topv = topv_v[...]

        def label_body(j, carry):
            pT_run, pT2_run = carry
            s = chunk_v[pl.ds(j * 16, 16)]
            meT = (s == T)
            meT2 = (s == T2)
            ceT = plsc.cumsum(jnp.where(meT, 1, 0))
            ceT2 = plsc.cumsum(jnp.where(meT2, 1, 0))
            rT = pT_run + ceT - jnp.where(meT, 1, 0)
            rT2 = pT2_run + ceT2 - jnp.where(meT2, 1, 0)
            in_top = (s > T) | (meT & (rT < limT))
            in_bot = (s < T2) | (meT2 & (rT2 < limT2))
            out_v[pl.ds(j * 16, 16)] = jnp.where(
                in_top & (~in_bot), topv, jnp.float32(0.0))
            return pT_run + jnp.max(ceT), pT2_run + jnp.max(ceT2)

        lax.fori_loop(0, NV, label_body, (my_prefT + z16, my_prefT2 + z16), unroll=2)

        @pl.when(tid < NT - 1)
        def _wfull():
            pltpu.sync_copy(out_v, out_hbm.at[pl.ds(base, CHUNK)])

        @pl.when(tid == NT - 1)
        def _wtail():
            pltpu.sync_copy(out_v.at[pl.ds(0, TAIL)],
                            out_hbm.at[pl.ds((NT - 1) * CHUNK, TAIL)])


@jax.jit
def _run(instances, bag_label, W1, b1, W2, b2):
    x = instances[0]
    preds = pl.pallas_call(
        _mlp_kernel,
        grid=(N_INST // BN,),
        in_specs=[
            pl.BlockSpec((BN, D_FEAT), lambda i: (i, 0)),
            pl.BlockSpec((D_FEAT, D_HID), lambda i: (0, 0)),
            pl.BlockSpec((1, D_HID), lambda i: (0, 0)),
            pl.BlockSpec((D_HID, 1), lambda i: (0, 0)),
            pl.BlockSpec((1, 1), lambda i: (0, 0)),
        ],
        out_specs=pl.BlockSpec((BN, 1), lambda i: (i, 0)),
        out_shape=jax.ShapeDtypeStruct((N_INST, 1), jnp.float32),
    )(x, W1, b1.reshape(1, D_HID), W2, b2.reshape(1, 1))

    top_val = jnp.where(bag_label[0] != 0.0, jnp.float32(1.0), jnp.float32(0.0))
    topv = jnp.broadcast_to(top_val, (16,))

    mesh = plsc.VectorSubcoreMesh(core_axis_name="c", subcore_axis_name="s")
    sel = pl.kernel(
        _sc_select,
        mesh=mesh,
        compiler_params=pltpu.CompilerParams(needs_layout_passes=False),
        out_type=jax.ShapeDtypeStruct((N_INST,), jnp.float32),
        scratch_types=[
            pltpu.VMEM((CHUNK,), jnp.float32),
            pltpu.VMEM((CHUNK,), jnp.int32),
            pltpu.VMEM((16,), jnp.float32),
            pltpu.VMEM((512,), jnp.int32),
            pltpu.VMEM((NT * 512,), jnp.int32),
            pltpu.VMEM((16,), jnp.int32),
            pltpu.VMEM((NT * 16,), jnp.int32),
            pltpu.VMEM((16,), jnp.int32),
            pltpu.VMEM((CHUNK,), jnp.float32),
            pltpu.VMEM_SHARED((NT * 512,), jnp.int32),
            pltpu.VMEM_SHARED((NT * 16,), jnp.int32),
            pltpu.VMEM_SHARED((16,), jnp.int32),
        ],
    )
    labels = sel(preds[:, 0], topv)

    return preds[None, ...], labels[:, None][None, ...]


def kernel(instances, bag_label, W1, b1, W2, b2):
    return _run(instances, bag_label, W1, b1, W2, b2)
